# Initial kernel scaffold; baseline (speedup 1.0000x reference)
import jax
import jax.numpy as jnp
from jax.experimental import pallas as pl


def kernel(x, weight):
    raise NotImplementedError("write your pallas kernel here")


# speedup vs baseline: 1.8392x; 1.8392x over previous
import functools

import jax
import jax.numpy as jnp
from jax import lax
from jax.experimental import pallas as pl
from jax.experimental.pallas import tpu as pltpu
from jax.experimental.pallas import tpu_sc as plsc

NUM_EMBEDDINGS = 1000000
EMBEDDING_DIM = 64

_info = plsc.get_sparse_core_info()
NC, NS = _info.num_cores, _info.num_subcores
NW = NC * NS

B_TOTAL = 16384 * 50
CHUNK = 128
N_CHUNKS = B_TOTAL // (NW * CHUNK)
B_PER_W = N_CHUNKS * CHUNK


def _gather_kernel(idx_hbm, table_hbm, out_hbm,
                   idx_v, rows_a, rows_b, sem_a, sem_b):
    wid = lax.axis_index("s") * NC + lax.axis_index("c")
    base = wid * B_PER_W
    pltpu.sync_copy(idx_hbm.at[wid], idx_v)

    pltpu.async_copy(table_hbm.at[idx_v.at[0]], rows_a, sem_a)

    def step(j, rows, sem, rows_next, sem_next):
        pltpu.async_copy(table_hbm.at[idx_v.at[j + 1]], rows_next, sem_next)
        pltpu.make_async_copy(table_hbm.at[idx_v.at[j]], rows, sem).wait()
        pltpu.sync_copy(rows, out_hbm.at[pl.ds(base + j * CHUNK, CHUNK)])

    def body(j, carry):
        even = lax.rem(j, 2) == 0

        @pl.when(even)
        def _even_step():
            step(j, rows_a, sem_a, rows_b, sem_b)

        @pl.when(jnp.logical_not(even))
        def _odd_step():
            step(j, rows_b, sem_b, rows_a, sem_a)

        return carry

    lax.fori_loop(0, N_CHUNKS - 1, body, 0, unroll=False)

    j_last = N_CHUNKS - 1
    last_rows = rows_a if j_last % 2 == 0 else rows_b
    last_sem = sem_a if j_last % 2 == 0 else sem_b
    pltpu.make_async_copy(table_hbm.at[idx_v.at[j_last]], last_rows, last_sem).wait()
    pltpu.sync_copy(last_rows, out_hbm.at[pl.ds(base + j_last * CHUNK, CHUNK)])


@jax.jit
def _embedding_lookup(x, weight):
    idx = x.reshape(NW, N_CHUNKS, CHUNK).astype(jnp.int32)
    mesh = plsc.VectorSubcoreMesh(core_axis_name="c", subcore_axis_name="s")
    out = pl.kernel(
        _gather_kernel,
        mesh=mesh,
        out_type=jax.ShapeDtypeStruct((B_TOTAL, EMBEDDING_DIM), jnp.float32),
        scratch_types=[
            pltpu.VMEM((N_CHUNKS, CHUNK), jnp.int32),
            pltpu.VMEM((CHUNK, EMBEDDING_DIM), jnp.float32),
            pltpu.VMEM((CHUNK, EMBEDDING_DIM), jnp.float32),
            pltpu.SemaphoreType.DMA,
            pltpu.SemaphoreType.DMA,
        ],
        compiler_params=pltpu.CompilerParams(use_tc_tiling_on_sc=False),
    )(idx, weight)
    return out.reshape(x.shape + (EMBEDDING_DIM,))


def kernel(x, weight):
    return _embedding_lookup(x, weight)


# speedup vs baseline: 1.8751x; 1.0195x over previous
import functools

import jax
import jax.numpy as jnp
from jax import lax
from jax.experimental import pallas as pl
from jax.experimental.pallas import tpu as pltpu
from jax.experimental.pallas import tpu_sc as plsc

NUM_EMBEDDINGS = 1000000
EMBEDDING_DIM = 64

_info = plsc.get_sparse_core_info()
NC, NS = _info.num_cores, _info.num_subcores
NW = NC * NS

B_TOTAL = 16384 * 50
CHUNK = 512
N_CHUNKS = B_TOTAL // (NW * CHUNK)
B_PER_W = N_CHUNKS * CHUNK


def _gather_kernel(idx_hbm, table_hbm, out_hbm,
                   idx_v, rows_a, rows_b, sem_a, sem_b):
    wid = lax.axis_index("s") * NC + lax.axis_index("c")
    base = wid * B_PER_W
    pltpu.sync_copy(idx_hbm.at[wid], idx_v)

    pltpu.async_copy(table_hbm.at[idx_v.at[0]], rows_a, sem_a)

    def step(j, rows, sem, rows_next, sem_next):
        pltpu.async_copy(table_hbm.at[idx_v.at[j + 1]], rows_next, sem_next)
        pltpu.make_async_copy(table_hbm.at[idx_v.at[j]], rows, sem).wait()
        pltpu.sync_copy(rows, out_hbm.at[pl.ds(base + j * CHUNK, CHUNK)])

    def body(j, carry):
        even = lax.rem(j, 2) == 0

        @pl.when(even)
        def _even_step():
            step(j, rows_a, sem_a, rows_b, sem_b)

        @pl.when(jnp.logical_not(even))
        def _odd_step():
            step(j, rows_b, sem_b, rows_a, sem_a)

        return carry

    lax.fori_loop(0, N_CHUNKS - 1, body, 0, unroll=False)

    j_last = N_CHUNKS - 1
    last_rows = rows_a if j_last % 2 == 0 else rows_b
    last_sem = sem_a if j_last % 2 == 0 else sem_b
    pltpu.make_async_copy(table_hbm.at[idx_v.at[j_last]], last_rows, last_sem).wait()
    pltpu.sync_copy(last_rows, out_hbm.at[pl.ds(base + j_last * CHUNK, CHUNK)])


@jax.jit
def _embedding_lookup(x, weight):
    idx = x.reshape(NW, N_CHUNKS, CHUNK).astype(jnp.int32)
    mesh = plsc.VectorSubcoreMesh(core_axis_name="c", subcore_axis_name="s")
    out = pl.kernel(
        _gather_kernel,
        mesh=mesh,
        out_type=jax.ShapeDtypeStruct((B_TOTAL, EMBEDDING_DIM), jnp.float32),
        scratch_types=[
            pltpu.VMEM((N_CHUNKS, CHUNK), jnp.int32),
            pltpu.VMEM((CHUNK, EMBEDDING_DIM), jnp.float32),
            pltpu.VMEM((CHUNK, EMBEDDING_DIM), jnp.float32),
            pltpu.SemaphoreType.DMA,
            pltpu.SemaphoreType.DMA,
        ],
        compiler_params=pltpu.CompilerParams(use_tc_tiling_on_sc=False),
    )(idx, weight)
    return out.reshape(x.shape + (EMBEDDING_DIM,))


def kernel(x, weight):
    return _embedding_lookup(x, weight)
